# baseline (speedup 1.0000x reference)
import functools

import jax
import jax.numpy as jnp
from jax import lax
from jax.experimental import pallas as pl
from jax.experimental.pallas import tpu as pltpu
from jax.experimental.pallas import tpu_sc as plsc


def kernel(num_M, num_VN, IVF, VN_index, wout):
    M, K, V = IVF.shape
    scale = 1.0 / M
    idx = VN_index.astype(jnp.int32)
    i0s = [idx[0, v] for v in range(V)]
    i1s = [idx[1, v] for v in range(V)]
    mesh = plsc.ScalarSubcoreMesh(axis_name="c", num_cores=1)

    @functools.partial(
        pl.kernel,
        mesh=mesh,
        compiler_params=pltpu.CompilerParams(
            needs_layout_passes=False,
            disable_bounds_checks=True,
            disable_semaphore_checks=True,
            skip_device_barrier=True,
        ),
        out_type=jax.ShapeDtypeStruct((M, V), jnp.float32),
        scratch_types=[
            pltpu.SMEM((M, K, V), jnp.float32),
            pltpu.SMEM((M, V), jnp.float32),
            pltpu.SMEM((M, V), jnp.float32),
            pltpu.SemaphoreType.DMA,
        ],
    )
    def scs_kernel(ivf_hbm, wout_hbm, out_hbm, ivf_s, wout_s, out_s, sem):
        c1 = pltpu.make_async_copy(ivf_hbm, ivf_s, sem)
        c2 = pltpu.make_async_copy(wout_hbm, wout_s, sem)
        c1.start()
        c2.start()
        c1.wait()
        c2.wait()
        for v in range(V):
            i0 = i0s[v]
            i1 = i1s[v]
            for m in range(M):
                a = ivf_s[m, i0, v]
                b = ivf_s[m, i1, v]
                out_s[m, v] = scale * a * b * wout_s[m, v]
        pltpu.sync_copy(out_s, out_hbm)

    return scs_kernel(IVF.astype(jnp.float32), wout.astype(jnp.float32))
